# baseline (speedup 1.0000x reference)
import jax
import jax.numpy as jnp
from jax.experimental import pallas as pl

_K = 16


def _body(x_ref, z_ref, w_ref, s_ref, o_ref):
    x = x_ref[...]
    z = z_ref[...]
    w = w_ref[...]
    s = s_ref[...]
    ut = jax.lax.dot_general(w, x, (((1,), (1,)), ((), ())),
                             preferred_element_type=jnp.float32)
    ut = ut + jax.lax.dot_general(s, z, (((1,), (1,)), ((), ())),
                                  preferred_element_type=jnp.float32)
    b = jnp.abs(ut)
    m = None
    for i in range(_K):
        m = jnp.max(b, axis=0, keepdims=True)
        if i < _K - 1:
            b = jnp.where(b >= m, -1.0, b)
    u = jax.lax.dot_general(x, w, (((1,), (1,)), ((), ())),
                            preferred_element_type=jnp.float32)
    u = u + jax.lax.dot_general(z, s, (((1,), (1,)), ((), ())),
                                preferred_element_type=jnp.float32)
    tcol = jax.lax.transpose(m, (1, 0))
    o_ref[...] = jnp.where(jnp.abs(u) >= tcol, u, 0.0)


@jax.jit
def kernel(x, z_prev, W, S):
    n, d_in = x.shape
    code = W.shape[0]
    blk = 2048
    return pl.pallas_call(
        _body,
        grid=(n // blk,),
        in_specs=[
            pl.BlockSpec((blk, d_in), lambda i: (i, 0)),
            pl.BlockSpec((blk, code), lambda i: (i, 0)),
            pl.BlockSpec((code, d_in), lambda i: (0, 0)),
            pl.BlockSpec((code, code), lambda i: (0, 0)),
        ],
        out_specs=pl.BlockSpec((blk, code), lambda i: (i, 0)),
        out_shape=jax.ShapeDtypeStruct((n, code), jnp.float32),
    )(x, z_prev, W, S)


# speedup vs baseline: 1.1369x; 1.1369x over previous
import jax
import jax.numpy as jnp
from jax.experimental import pallas as pl

_K = 16


def _body(x_ref, z_ref, w_ref, s_ref, o_ref):
    x = x_ref[...]
    z = z_ref[...]
    w = w_ref[...]
    s = s_ref[...]
    ut = jax.lax.dot_general(w, x, (((1,), (1,)), ((), ())),
                             preferred_element_type=jnp.float32)
    ut = ut + jax.lax.dot_general(s, z, (((1,), (1,)), ((), ())),
                                  preferred_element_type=jnp.float32)
    a = jnp.abs(ut)
    b = a
    m = None
    for i in range(_K):
        m = jnp.max(b, axis=0, keepdims=True)
        if i < _K - 1:
            b = jnp.where(b >= m, -1.0, b)
    masked = jnp.where(a >= m, ut, 0.0)
    o_ref[...] = jax.lax.transpose(masked, (1, 0))


@jax.jit
def kernel(x, z_prev, W, S):
    n, d_in = x.shape
    code = W.shape[0]
    blk = 4096
    return pl.pallas_call(
        _body,
        grid=(n // blk,),
        in_specs=[
            pl.BlockSpec((blk, d_in), lambda i: (i, 0)),
            pl.BlockSpec((blk, code), lambda i: (i, 0)),
            pl.BlockSpec((code, d_in), lambda i: (0, 0)),
            pl.BlockSpec((code, code), lambda i: (0, 0)),
        ],
        out_specs=pl.BlockSpec((blk, code), lambda i: (i, 0)),
        out_shape=jax.ShapeDtypeStruct((n, code), jnp.float32),
    )(x, z_prev, W, S)
